# Initial kernel scaffold; baseline (speedup 1.0000x reference)
import jax
import jax.numpy as jnp
from jax.experimental import pallas as pl


def kernel(node_feats, edge_index, W_edge, b_edge, W_proj, b_proj, W_ih, b_ih, W_hh, b_hh, gamma, beta):
    raise NotImplementedError("write your pallas kernel here")


# speedup vs baseline: 18.5318x; 18.5318x over previous
import dataclasses
import functools

import jax
import jax.numpy as jnp
from jax import lax
from jax.experimental import pallas as pl
from jax.experimental.pallas import tpu as pltpu
from jax.experimental.pallas import tpu_sc as plsc

N = 10000
E = 320000
D = 128
G = 128

NP = 10016
NPD = 10240
EP = 327680
ER = EP // 128
ROWS_A = ER // 16
ROWS_C = ER // 32
DEN0 = 10080

_HIGH = jax.lax.Precision.HIGHEST


def _dot(a, b):
    return jax.lax.dot_general(a, b, (((1,), (0,)), ((), ())),
                               precision=_HIGH, preferred_element_type=jnp.float32)


def _prep_body(x_ref, wcat_ref, bp_ref, hv_ref, pq_ref):
    acc = _dot(x_ref[...], wcat_ref[...])
    hv_ref[...] = acc[:, :G] + bp_ref[...]
    pq_ref[...] = acc[:, G:]


def _prep(x, wcat, bp):
    blk = 2000
    return pl.pallas_call(
        _prep_body,
        grid=(N // blk,),
        in_specs=[
            pl.BlockSpec((blk, D), lambda i: (i, 0)),
            pl.BlockSpec((D, 2 * G), lambda i: (0, 0)),
            pl.BlockSpec((1, G), lambda i: (0, 0)),
        ],
        out_specs=[
            pl.BlockSpec((blk, G), lambda i: (i, 0)),
            pl.BlockSpec((blk, G), lambda i: (i, 0)),
        ],
        out_shape=[
            jax.ShapeDtypeStruct((N, G), jnp.float32),
            jax.ShapeDtypeStruct((N, G), jnp.float32),
        ],
    )(x, wcat, bp)


def _sc_body(pd_h, ps_h, hv_h, src_h, dst_h, b_h, cout_h,
             pd_v, ps_v, src_v, dst_v, den_v, rows_v, w_v, idxr_v,
             c_sh):
    cid = lax.axis_index("core")
    sid = lax.axis_index("subcore")

    pltpu.sync_copy(pd_h, pd_v)
    pltpu.sync_copy(ps_h, ps_v)
    pltpu.sync_copy(b_h, w_v.at[pl.ds(112, 16)])

    zeros16 = jnp.zeros((16,), jnp.float32)

    @pl.loop(0, 80)
    def _(i):
        for k in range(8):
            den_v[i, pl.ds(k * 16, 16)] = zeros16

    @pl.loop(0, 128)
    def _(i):
        for k in range(8):
            rows_v[i, pl.ds(k * 16, 16)] = zeros16

    @pl.loop(0, 5)
    def _(i):
        pltpu.sync_copy(rows_v, c_sh.at[pl.ds(sid * 640 + i * 128, 128)])

    w_v[pl.ds(0, 16)] = jnp.full((16,), -3e38, jnp.float32)
    w_v[pl.ds(16, 16)] = jnp.full((16,), -3e38, jnp.float32)

    @pl.loop(0, NP // 16)
    def _(i):
        w_v[pl.ds(0, 16)] = jnp.maximum(w_v[pl.ds(0, 16)],
                                        pd_v[pl.ds(i * 16, 16)])
        w_v[pl.ds(16, 16)] = jnp.maximum(w_v[pl.ds(16, 16)],
                                         ps_v[pl.ds(i * 16, 16)])

    bsc = w_v[pl.ds(112, 16)][0]
    zm = jnp.max(w_v[pl.ds(0, 16)]) + jnp.max(w_v[pl.ds(16, 16)]) + bsc
    mtop = jnp.maximum(zm, 0.01 * zm)

    base_a = sid * ROWS_A

    with jax.named_scope("sc_phase_a"):
        @pl.loop(0, ROWS_A // 8)
        def _(c8):
            pltpu.sync_copy(src_h.at[pl.ds(base_a + c8 * 8, 8)], src_v)
            pltpu.sync_copy(dst_h.at[pl.ds(base_a + c8 * 8, 8)], dst_v)

            @pl.loop(0, 8)
            def _(r):
                for k in range(8):
                    si = src_v[r, pl.ds(k * 16, 16)]
                    di = dst_v[r, pl.ds(k * 16, 16)]
                    z = (plsc.load_gather(pd_v, [di])
                         + plsc.load_gather(ps_v, [si]) + bsc)
                    l = jnp.maximum(z, 0.01 * z)
                    ex = jnp.exp(l - mtop)
                    plsc.addupdate_scatter(
                        den_v,
                        [lax.shift_right_logical(di, 7),
                         lax.bitwise_and(di, 127)],
                        ex)

    with jax.named_scope("sc_combine"):
        for k in range(5):
            idxr_v[0, pl.ds(k * 16, 16)] = (lax.iota(jnp.int32, 16)
                                            + (DEN0 + k * 16))

        plsc.subcore_barrier()
        pltpu.sync_copy(den_v, c_sh.at[idxr_v.at[0]], add=True)
        plsc.subcore_barrier()

        @pl.loop(0, 10)
        def _(i):
            pltpu.sync_copy(c_sh.at[pl.ds(DEN0 + i * 8, 8)],
                            den_v.at[pl.ds(i * 8, 8)])

    wid = cid * 16 + sid
    base_c = wid * ROWS_C

    with jax.named_scope("sc_phase_c"):
        @pl.loop(0, ROWS_C // 8)
        def _(cb):
            pltpu.sync_copy(src_h.at[pl.ds(base_c + cb * 8, 8)], src_v)
            pltpu.sync_copy(dst_h.at[pl.ds(base_c + cb * 8, 8)], dst_v)

            @pl.loop(0, 8)
            def _(r):
                pltpu.sync_copy(hv_h.at[src_v.at[r]], rows_v)
                for k in range(8):
                    si = src_v[r, pl.ds(k * 16, 16)]
                    di = dst_v[r, pl.ds(k * 16, 16)]
                    z = (plsc.load_gather(pd_v, [di])
                         + plsc.load_gather(ps_v, [si]) + bsc)
                    l = jnp.maximum(z, 0.01 * z)
                    ex = jnp.exp(l - mtop)
                    den = plsc.load_gather(
                        den_v,
                        [lax.shift_right_logical(di, 7),
                         lax.bitwise_and(di, 127)])
                    w_v[pl.ds(k * 16, 16)] = ex / den + 1.0

                @pl.loop(0, 8)
                def _(kc):
                    wch = w_v[pl.ds(kc * 16, 16)]
                    for lane in range(16):
                        ws = wch[lane]
                        e = kc * 16 + lane
                        for m in range(8):
                            rows_v[e, pl.ds(m * 16, 16)] = (
                                rows_v[e, pl.ds(m * 16, 16)] * ws)

                pltpu.sync_copy(rows_v, c_sh.at[dst_v.at[r]], add=True)

    with jax.named_scope("sc_copyout"):
        plsc.subcore_barrier()

        @pl.loop(0, 5)
        def _(i):
            pltpu.sync_copy(c_sh.at[pl.ds(sid * 640 + i * 128, 128)], rows_v)
            pltpu.sync_copy(
                rows_v,
                cout_h.at[pl.ds(cid * NPD + sid * 640 + i * 128, 128)])


def _sc_aggregate(pd, ps, hv_p, src_p, dst_p, b16):
    mesh = plsc.VectorSubcoreMesh(core_axis_name="core", subcore_axis_name="subcore")
    cp = pltpu.CompilerParams()
    if "needs_layout_passes" in pltpu.CompilerParams.__dataclass_fields__:
        cp = dataclasses.replace(cp, needs_layout_passes=False)
    return pl.kernel(
        _sc_body,
        compiler_params=cp,
        out_type=jax.ShapeDtypeStruct((2 * NPD, G), jnp.float32),
        mesh=mesh,
        scratch_types=[
            pltpu.VMEM((NP,), jnp.float32),
            pltpu.VMEM((NP,), jnp.float32),
            pltpu.VMEM((8, 128), jnp.int32),
            pltpu.VMEM((8, 128), jnp.int32),
            pltpu.VMEM((80, 128), jnp.float32),
            pltpu.VMEM((128, G), jnp.float32),
            pltpu.VMEM((128,), jnp.float32),
            pltpu.VMEM((1, 80), jnp.int32),
            pltpu.VMEM_SHARED((NPD, G), jnp.float32),
        ],
    )(pd, ps, hv_p, src_p, dst_p, b16)


def _gru_body(cp_ref, x_ref, wih_ref, whh_ref, bih_ref, bhh_ref, g_ref, bt_ref,
              y_ref, out_scr, sum_scr, sq_scr):
    p = pl.program_id(0)
    i = pl.program_id(1)
    blk = 1000

    @pl.when(p == 0)
    def _():
        xb = x_ref[...]
        cb = cp_ref[0] + cp_ref[1]
        ctx = jnp.where(cb > 0, cb, jnp.exp(jnp.minimum(cb, 0.0)) - 1.0)
        gi = _dot(ctx, wih_ref[...]) + bih_ref[...]
        gh = _dot(xb, whh_ref[...]) + bhh_ref[...]
        r = jax.nn.sigmoid(gi[:, :G] + gh[:, :G])
        z = jax.nn.sigmoid(gi[:, G:2 * G] + gh[:, G:2 * G])
        n = jnp.tanh(gi[:, 2 * G:] + r * gh[:, 2 * G:])
        out = jnp.maximum((1.0 - z) * n + z * xb, 0.0)
        out_scr[pl.ds(i * blk, blk), :] = out
        so = jnp.sum(out, axis=0, keepdims=True)
        sq = jnp.sum(out * out, axis=0, keepdims=True)

        @pl.when(i == 0)
        def _():
            sum_scr[...] = so
            sq_scr[...] = sq

        @pl.when(i > 0)
        def _():
            sum_scr[...] += so
            sq_scr[...] += sq

    @pl.when(p == 1)
    def _():
        mean = sum_scr[...] * (1.0 / N)
        var = sq_scr[...] * (1.0 / N) - mean * mean
        inv = jax.lax.rsqrt(var + 1e-5)
        y_ref[...] = ((out_scr[pl.ds(i * blk, blk), :] - mean) * inv
                      * g_ref[...] + bt_ref[...])


def _gru_bn(c_pair, x, wih, whh, bih, bhh, gamma, beta):
    blk = 1000
    return pl.pallas_call(
        _gru_body,
        grid=(2, N // blk),
        in_specs=[
            pl.BlockSpec((2, blk, G), lambda p, i: (0, i, 0)),
            pl.BlockSpec((blk, D), lambda p, i: (i, 0)),
            pl.BlockSpec((G, 3 * D), lambda p, i: (0, 0)),
            pl.BlockSpec((D, 3 * D), lambda p, i: (0, 0)),
            pl.BlockSpec((1, 3 * D), lambda p, i: (0, 0)),
            pl.BlockSpec((1, 3 * D), lambda p, i: (0, 0)),
            pl.BlockSpec((1, G), lambda p, i: (0, 0)),
            pl.BlockSpec((1, G), lambda p, i: (0, 0)),
        ],
        out_specs=pl.BlockSpec((blk, G), lambda p, i: (i, 0)),
        out_shape=jax.ShapeDtypeStruct((N, G), jnp.float32),
        scratch_shapes=[
            pltpu.VMEM((N, G), jnp.float32),
            pltpu.VMEM((1, G), jnp.float32),
            pltpu.VMEM((1, G), jnp.float32),
        ],
    )(c_pair, x, wih, whh, bih, bhh, gamma, beta)


def kernel(node_feats, edge_index, W_edge, b_edge, W_proj, b_proj,
           W_ih, b_ih, W_hh, b_hh, gamma, beta):
    wcat = jnp.concatenate(
        [W_proj.T,
         W_edge[0, :D][:, None],
         W_edge[0, D:][:, None],
         jnp.zeros((D, G - 2), jnp.float32)], axis=1)
    hv, pq = _prep(node_feats, wcat, b_proj.reshape(1, G))

    pd = jnp.pad(pq[:, 0], (0, NP - N))
    ps = jnp.pad(pq[:, 1], (0, NP - N))
    hv_p = jnp.pad(hv, ((0, NP - N), (0, 0)))

    pad_n = (EP - E) // 32
    pad_src = jnp.broadcast_to(
        N + (jnp.arange(pad_n, dtype=jnp.int32) % 16), (32, pad_n))
    pad_dst = jnp.broadcast_to(
        N + (jnp.arange(32, dtype=jnp.int32) % 16)[:, None], (32, pad_n))
    src_p = jnp.concatenate(
        [edge_index[0].reshape(32, E // 32), pad_src], axis=1).reshape(ER, 128)
    dst_p = jnp.concatenate(
        [edge_index[1].reshape(32, E // 32), pad_dst], axis=1).reshape(ER, 128)
    b16 = jnp.broadcast_to(b_edge, (16,)).astype(jnp.float32)

    cout = _sc_aggregate(pd, ps, hv_p, src_p, dst_p, b16)
    c_pair = cout.reshape(2, NPD, G)

    return _gru_bn(c_pair, node_feats,
                   W_ih.T, W_hh.T,
                   b_ih.reshape(1, 3 * D), b_hh.reshape(1, 3 * D),
                   gamma.reshape(1, G), beta.reshape(1, G))


# speedup vs baseline: 21.8904x; 1.1812x over previous
import dataclasses
import functools

import jax
import jax.numpy as jnp
from jax import lax
from jax.experimental import pallas as pl
from jax.experimental.pallas import tpu as pltpu
from jax.experimental.pallas import tpu_sc as plsc

N = 10000
E = 320000
D = 128
G = 128

NP = 10016
NPD = 10240
EP = 327680
ER = EP // 128
ROWS_A = ER // 16
ROWS_C = ER // 32
DEN0 = 10080

_HIGH = jax.lax.Precision.HIGHEST


def _dot(a, b):
    return jax.lax.dot_general(a, b, (((1,), (0,)), ((), ())),
                               precision=_HIGH, preferred_element_type=jnp.float32)


def _prep_body(x_ref, wcat_ref, bp_ref, hv_ref, pq_ref):
    acc = _dot(x_ref[...], wcat_ref[...])
    hv_ref[...] = acc[:, :G] + bp_ref[...]
    pq_ref[...] = acc[:, G:]


def _prep(x, wcat, bp):
    blk = 2000
    return pl.pallas_call(
        _prep_body,
        grid=(N // blk,),
        in_specs=[
            pl.BlockSpec((blk, D), lambda i: (i, 0)),
            pl.BlockSpec((D, 2 * G), lambda i: (0, 0)),
            pl.BlockSpec((1, G), lambda i: (0, 0)),
        ],
        out_specs=[
            pl.BlockSpec((blk, G), lambda i: (i, 0)),
            pl.BlockSpec((blk, G), lambda i: (i, 0)),
        ],
        out_shape=[
            jax.ShapeDtypeStruct((N, G), jnp.float32),
            jax.ShapeDtypeStruct((N, G), jnp.float32),
        ],
    )(x, wcat, bp)


def _sc_body(pd_h, ps_h, hv_h, src_h, dst_h, b_h, cout_h,
             pd_v, ps_v, src_v, dst_v, den_v, rows_v, w_v, idxr_v,
             c_sh, gs0, gs1, ss0, ss1):
    cid = lax.axis_index("core")
    sid = lax.axis_index("subcore")
    gs = (gs0, gs1)
    ss = (ss0, ss1)

    pltpu.sync_copy(pd_h, pd_v)
    pltpu.sync_copy(ps_h, ps_v)
    pltpu.sync_copy(b_h, w_v.at[pl.ds(112, 16)])

    zeros16 = jnp.zeros((16,), jnp.float32)

    @pl.loop(0, 80)
    def _(i):
        for k in range(8):
            den_v[i, pl.ds(k * 16, 16)] = zeros16

    @pl.loop(0, 128)
    def _(i):
        for k in range(8):
            rows_v[i, pl.ds(k * 16, 16)] = zeros16

    @pl.loop(0, 5)
    def _(i):
        pltpu.sync_copy(rows_v, c_sh.at[pl.ds(sid * 640 + i * 128, 128)])

    w_v[pl.ds(0, 16)] = jnp.full((16,), -3e38, jnp.float32)
    w_v[pl.ds(16, 16)] = jnp.full((16,), -3e38, jnp.float32)

    @pl.loop(0, NP // 16)
    def _(i):
        w_v[pl.ds(0, 16)] = jnp.maximum(w_v[pl.ds(0, 16)],
                                        pd_v[pl.ds(i * 16, 16)])
        w_v[pl.ds(16, 16)] = jnp.maximum(w_v[pl.ds(16, 16)],
                                         ps_v[pl.ds(i * 16, 16)])

    bsc = w_v[pl.ds(112, 16)][0]
    zm = jnp.max(w_v[pl.ds(0, 16)]) + jnp.max(w_v[pl.ds(16, 16)]) + bsc
    mtop = jnp.maximum(zm, 0.01 * zm)

    base_a = sid * ROWS_A

    with jax.named_scope("sc_phase_a"):
        @pl.loop(0, ROWS_A // 8)
        def _(c8):
            pltpu.sync_copy(src_h.at[pl.ds(base_a + c8 * 8, 8)], src_v)
            pltpu.sync_copy(dst_h.at[pl.ds(base_a + c8 * 8, 8)], dst_v)

            @pl.loop(0, 8)
            def _(r):
                for k in range(8):
                    si = src_v[r, pl.ds(k * 16, 16)]
                    di = dst_v[r, pl.ds(k * 16, 16)]
                    z = (plsc.load_gather(pd_v, [di])
                         + plsc.load_gather(ps_v, [si]) + bsc)
                    l = jnp.maximum(z, 0.01 * z)
                    ex = jnp.exp(l - mtop)
                    plsc.addupdate_scatter(
                        den_v,
                        [lax.shift_right_logical(di, 7),
                         lax.bitwise_and(di, 127)],
                        ex)

    with jax.named_scope("sc_combine"):
        for k in range(5):
            idxr_v[0, pl.ds(k * 16, 16)] = (lax.iota(jnp.int32, 16)
                                            + (DEN0 + k * 16))

        plsc.subcore_barrier()
        pltpu.sync_copy(den_v, c_sh.at[idxr_v.at[0]], add=True)
        plsc.subcore_barrier()

        @pl.loop(0, 10)
        def _(i):
            pltpu.sync_copy(c_sh.at[pl.ds(DEN0 + i * 8, 8)],
                            den_v.at[pl.ds(i * 8, 8)])

    wid = cid * 16 + sid
    base_c = wid * ROWS_C

    half = (rows_v.at[pl.ds(0, 64)], rows_v.at[pl.ds(64, 64)])

    def _gidx(r, h):
        return src_v.at[r, pl.ds(h * 64, 64)]

    def _sidx(r, h):
        return dst_v.at[r, pl.ds(h * 64, 64)]

    def _scale_half(r, h):
        for k in range(4):
            si = src_v[r, pl.ds(h * 64 + k * 16, 16)]
            di = dst_v[r, pl.ds(h * 64 + k * 16, 16)]
            z = (plsc.load_gather(pd_v, [di])
                 + plsc.load_gather(ps_v, [si]) + bsc)
            l = jnp.maximum(z, 0.01 * z)
            ex = jnp.exp(l - mtop)
            den = plsc.load_gather(
                den_v,
                [lax.shift_right_logical(di, 7), lax.bitwise_and(di, 127)])
            w_v[pl.ds(k * 16, 16)] = ex / den + 1.0

        @pl.loop(0, 4)
        def _(kc):
            wch = w_v[pl.ds(kc * 16, 16)]
            for lane in range(16):
                ws = wch[lane]
                e = h * 64 + kc * 16 + lane
                for m in range(8):
                    rows_v[e, pl.ds(m * 16, 16)] = (
                        rows_v[e, pl.ds(m * 16, 16)] * ws)

    with jax.named_scope("sc_phase_c"):
        pltpu.sync_copy(src_h.at[pl.ds(base_c, 8)], src_v)
        pltpu.sync_copy(dst_h.at[pl.ds(base_c, 8)], dst_v)
        pltpu.async_copy(hv_h.at[_gidx(0, 0)], half[0], gs[0])
        pltpu.async_copy(hv_h.at[_gidx(0, 1)], half[1], gs[1])

        @pl.loop(0, ROWS_C)
        def _(q):
            r = q % 8
            for h in range(2):
                pltpu.make_async_copy(hv_h.at[_gidx(r, h)],
                                      half[h], gs[h]).wait()
                _scale_half(r, h)
                pltpu.async_copy(half[h], c_sh.at[_sidx(r, h)], ss[h],
                                 add=True)

            @pl.when(q < ROWS_C - 1)
            def _():
                @pl.when(r == 7)
                def _():
                    for h in range(2):
                        pltpu.make_async_copy(half[h], c_sh.at[_sidx(7, h)],
                                              ss[h]).wait()
                    nxt = base_c + (q // 8 + 1) * 8
                    pltpu.sync_copy(src_h.at[pl.ds(nxt, 8)], src_v)
                    pltpu.sync_copy(dst_h.at[pl.ds(nxt, 8)], dst_v)
                    for h in range(2):
                        pltpu.async_copy(hv_h.at[_gidx(0, h)], half[h], gs[h])

                @pl.when(r < 7)
                def _():
                    for h in range(2):
                        pltpu.make_async_copy(half[h], c_sh.at[_sidx(r, h)],
                                              ss[h]).wait()
                        pltpu.async_copy(hv_h.at[_gidx(r + 1, h)],
                                         half[h], gs[h])

        for h in range(2):
            pltpu.make_async_copy(half[h], c_sh.at[_sidx(7, h)], ss[h]).wait()

    with jax.named_scope("sc_copyout"):
        plsc.subcore_barrier()

        @pl.loop(0, 5)
        def _(i):
            pltpu.sync_copy(c_sh.at[pl.ds(sid * 640 + i * 128, 128)], rows_v)
            pltpu.sync_copy(
                rows_v,
                cout_h.at[pl.ds(cid * NPD + sid * 640 + i * 128, 128)])


def _sc_aggregate(pd, ps, hv_p, src_p, dst_p, b16):
    mesh = plsc.VectorSubcoreMesh(core_axis_name="core", subcore_axis_name="subcore")
    cp = pltpu.CompilerParams()
    if "needs_layout_passes" in pltpu.CompilerParams.__dataclass_fields__:
        cp = dataclasses.replace(cp, needs_layout_passes=False)
    return pl.kernel(
        _sc_body,
        compiler_params=cp,
        out_type=jax.ShapeDtypeStruct((2 * NPD, G), jnp.float32),
        mesh=mesh,
        scratch_types=[
            pltpu.VMEM((NP,), jnp.float32),
            pltpu.VMEM((NP,), jnp.float32),
            pltpu.VMEM((8, 128), jnp.int32),
            pltpu.VMEM((8, 128), jnp.int32),
            pltpu.VMEM((80, 128), jnp.float32),
            pltpu.VMEM((128, G), jnp.float32),
            pltpu.VMEM((128,), jnp.float32),
            pltpu.VMEM((1, 80), jnp.int32),
            pltpu.VMEM_SHARED((NPD, G), jnp.float32),
            pltpu.SemaphoreType.DMA,
            pltpu.SemaphoreType.DMA,
            pltpu.SemaphoreType.DMA,
            pltpu.SemaphoreType.DMA,
        ],
    )(pd, ps, hv_p, src_p, dst_p, b16)


def _gru_body(cp_ref, x_ref, wih_ref, whh_ref, bih_ref, bhh_ref, g_ref, bt_ref,
              y_ref, out_scr, sum_scr, sq_scr):
    p = pl.program_id(0)
    i = pl.program_id(1)
    blk = 1000

    @pl.when(p == 0)
    def _():
        xb = x_ref[...]
        cb = cp_ref[0] + cp_ref[1]
        ctx = jnp.where(cb > 0, cb, jnp.exp(jnp.minimum(cb, 0.0)) - 1.0)
        gi = _dot(ctx, wih_ref[...]) + bih_ref[...]
        gh = _dot(xb, whh_ref[...]) + bhh_ref[...]
        r = jax.nn.sigmoid(gi[:, :G] + gh[:, :G])
        z = jax.nn.sigmoid(gi[:, G:2 * G] + gh[:, G:2 * G])
        n = jnp.tanh(gi[:, 2 * G:] + r * gh[:, 2 * G:])
        out = jnp.maximum((1.0 - z) * n + z * xb, 0.0)
        out_scr[pl.ds(i * blk, blk), :] = out
        so = jnp.sum(out, axis=0, keepdims=True)
        sq = jnp.sum(out * out, axis=0, keepdims=True)

        @pl.when(i == 0)
        def _():
            sum_scr[...] = so
            sq_scr[...] = sq

        @pl.when(i > 0)
        def _():
            sum_scr[...] += so
            sq_scr[...] += sq

    @pl.when(p == 1)
    def _():
        mean = sum_scr[...] * (1.0 / N)
        var = sq_scr[...] * (1.0 / N) - mean * mean
        inv = jax.lax.rsqrt(var + 1e-5)
        y_ref[...] = ((out_scr[pl.ds(i * blk, blk), :] - mean) * inv
                      * g_ref[...] + bt_ref[...])


def _gru_bn(c_pair, x, wih, whh, bih, bhh, gamma, beta):
    blk = 1000
    return pl.pallas_call(
        _gru_body,
        grid=(2, N // blk),
        in_specs=[
            pl.BlockSpec((2, blk, G), lambda p, i: (0, i, 0)),
            pl.BlockSpec((blk, D), lambda p, i: (i, 0)),
            pl.BlockSpec((G, 3 * D), lambda p, i: (0, 0)),
            pl.BlockSpec((D, 3 * D), lambda p, i: (0, 0)),
            pl.BlockSpec((1, 3 * D), lambda p, i: (0, 0)),
            pl.BlockSpec((1, 3 * D), lambda p, i: (0, 0)),
            pl.BlockSpec((1, G), lambda p, i: (0, 0)),
            pl.BlockSpec((1, G), lambda p, i: (0, 0)),
        ],
        out_specs=pl.BlockSpec((blk, G), lambda p, i: (i, 0)),
        out_shape=jax.ShapeDtypeStruct((N, G), jnp.float32),
        scratch_shapes=[
            pltpu.VMEM((N, G), jnp.float32),
            pltpu.VMEM((1, G), jnp.float32),
            pltpu.VMEM((1, G), jnp.float32),
        ],
    )(c_pair, x, wih, whh, bih, bhh, gamma, beta)


def kernel(node_feats, edge_index, W_edge, b_edge, W_proj, b_proj,
           W_ih, b_ih, W_hh, b_hh, gamma, beta):
    wcat = jnp.concatenate(
        [W_proj.T,
         W_edge[0, :D][:, None],
         W_edge[0, D:][:, None],
         jnp.zeros((D, G - 2), jnp.float32)], axis=1)
    hv, pq = _prep(node_feats, wcat, b_proj.reshape(1, G))

    pd = jnp.pad(pq[:, 0], (0, NP - N))
    ps = jnp.pad(pq[:, 1], (0, NP - N))
    hv_p = jnp.pad(hv, ((0, NP - N), (0, 0)))

    pad_n = (EP - E) // 32
    pad_src = jnp.broadcast_to(
        N + (jnp.arange(pad_n, dtype=jnp.int32) % 16), (32, pad_n))
    pad_dst = jnp.broadcast_to(
        N + (jnp.arange(32, dtype=jnp.int32) % 16)[:, None], (32, pad_n))
    src_p = jnp.concatenate(
        [edge_index[0].reshape(32, E // 32), pad_src], axis=1).reshape(ER, 128)
    dst_p = jnp.concatenate(
        [edge_index[1].reshape(32, E // 32), pad_dst], axis=1).reshape(ER, 128)
    b16 = jnp.broadcast_to(b_edge, (16,)).astype(jnp.float32)

    cout = _sc_aggregate(pd, ps, hv_p, src_p, dst_p, b16)
    c_pair = cout.reshape(2, NPD, G)

    return _gru_bn(c_pair, node_feats,
                   W_ih.T, W_hh.T,
                   b_ih.reshape(1, 3 * D), b_hh.reshape(1, 3 * D),
                   gamma.reshape(1, G), beta.reshape(1, G))


# speedup vs baseline: 22.8419x; 1.0435x over previous
import dataclasses
import functools

import jax
import jax.numpy as jnp
from jax import lax
from jax.experimental import pallas as pl
from jax.experimental.pallas import tpu as pltpu
from jax.experimental.pallas import tpu_sc as plsc

N = 10000
E = 320000
D = 128
G = 128

NP = 10016
NPD = 10240
EP = 327680
ER = EP // 128
ROWS_A = ER // 16
ROWS_C = ER // 32
DEN0 = 10080

_HIGH = jax.lax.Precision.HIGHEST


def _dot(a, b):
    return jax.lax.dot_general(a, b, (((1,), (0,)), ((), ())),
                               precision=_HIGH, preferred_element_type=jnp.float32)


def _prep_body(x_ref, wcat_ref, bp_ref, hv_ref, pq_ref):
    acc = _dot(x_ref[...], wcat_ref[...])
    hv_ref[...] = acc[:, :G] + bp_ref[...]
    pq_ref[...] = acc[:, G:]


def _prep(x, wcat, bp):
    blk = 2000
    return pl.pallas_call(
        _prep_body,
        grid=(N // blk,),
        in_specs=[
            pl.BlockSpec((blk, D), lambda i: (i, 0)),
            pl.BlockSpec((D, 2 * G), lambda i: (0, 0)),
            pl.BlockSpec((1, G), lambda i: (0, 0)),
        ],
        out_specs=[
            pl.BlockSpec((blk, G), lambda i: (i, 0)),
            pl.BlockSpec((blk, G), lambda i: (i, 0)),
        ],
        out_shape=[
            jax.ShapeDtypeStruct((N, G), jnp.float32),
            jax.ShapeDtypeStruct((N, G), jnp.float32),
        ],
    )(x, wcat, bp)


def _sc_body(pd_h, ps_h, hv_h, src_h, dst_h, b_h, cout_h,
             pd_v, ps_v, src_v, dst_v, den_v, rows_v, w_v, idxr_v,
             c_sh, gs0, gs1, ss0, ss1):
    cid = lax.axis_index("core")
    sid = lax.axis_index("subcore")
    gs = (gs0, gs1)
    ss = (ss0, ss1)

    pltpu.sync_copy(pd_h, pd_v)
    pltpu.sync_copy(ps_h, ps_v)
    pltpu.sync_copy(b_h, w_v.at[pl.ds(112, 16)])

    zeros16 = jnp.zeros((16,), jnp.float32)

    @pl.loop(0, 80)
    def _(i):
        for k in range(8):
            den_v[i, pl.ds(k * 16, 16)] = zeros16

    @pl.loop(0, 128)
    def _(i):
        for k in range(8):
            rows_v[i, pl.ds(k * 16, 16)] = zeros16

    @pl.loop(0, 5)
    def _(i):
        pltpu.sync_copy(rows_v, c_sh.at[pl.ds(sid * 640 + i * 128, 128)])

    w_v[pl.ds(0, 16)] = jnp.full((16,), -3e38, jnp.float32)
    w_v[pl.ds(16, 16)] = jnp.full((16,), -3e38, jnp.float32)

    @pl.loop(0, NP // 16)
    def _(i):
        w_v[pl.ds(0, 16)] = jnp.maximum(w_v[pl.ds(0, 16)],
                                        pd_v[pl.ds(i * 16, 16)])
        w_v[pl.ds(16, 16)] = jnp.maximum(w_v[pl.ds(16, 16)],
                                         ps_v[pl.ds(i * 16, 16)])

    bsc = w_v[pl.ds(112, 16)][0]
    zm = jnp.max(w_v[pl.ds(0, 16)]) + jnp.max(w_v[pl.ds(16, 16)]) + bsc
    mtop = jnp.maximum(zm, 0.01 * zm)

    base_a = sid * ROWS_A

    with jax.named_scope("sc_phase_a"):
        @pl.loop(0, ROWS_A // 8)
        def _(c8):
            pltpu.async_copy(src_h.at[pl.ds(base_a + c8 * 8, 8)], src_v, gs0)
            pltpu.async_copy(dst_h.at[pl.ds(base_a + c8 * 8, 8)], dst_v, gs1)
            pltpu.make_async_copy(src_h.at[pl.ds(base_a + c8 * 8, 8)],
                                  src_v, gs0).wait()
            pltpu.make_async_copy(dst_h.at[pl.ds(base_a + c8 * 8, 8)],
                                  dst_v, gs1).wait()

            @pl.loop(0, 8)
            def _(r):
                for k in range(8):
                    si = src_v[r, pl.ds(k * 16, 16)]
                    di = dst_v[r, pl.ds(k * 16, 16)]
                    z = (plsc.load_gather(pd_v, [di])
                         + plsc.load_gather(ps_v, [si]) + bsc)
                    l = jnp.maximum(z, 0.01 * z)
                    ex = jnp.exp(l - mtop)
                    plsc.addupdate_scatter(
                        den_v,
                        [lax.shift_right_logical(di, 7),
                         lax.bitwise_and(di, 127)],
                        ex)

    with jax.named_scope("sc_combine"):
        for k in range(5):
            idxr_v[0, pl.ds(k * 16, 16)] = (lax.iota(jnp.int32, 16)
                                            + (DEN0 + k * 16))

        plsc.subcore_barrier()
        pltpu.sync_copy(den_v, c_sh.at[idxr_v.at[0]], add=True)
        plsc.subcore_barrier()

        @pl.loop(0, 10)
        def _(i):
            pltpu.sync_copy(c_sh.at[pl.ds(DEN0 + i * 8, 8)],
                            den_v.at[pl.ds(i * 8, 8)])

    wid = cid * 16 + sid
    base_c = wid * ROWS_C

    half = (rows_v.at[pl.ds(0, 64)], rows_v.at[pl.ds(64, 64)])

    def _gidx(r, h):
        return src_v.at[r, pl.ds(h * 64, 64)]

    def _sidx(r, h):
        return dst_v.at[r, pl.ds(h * 64, 64)]

    def _scale_half(r, h):
        for k in range(4):
            si = src_v[r, pl.ds(h * 64 + k * 16, 16)]
            di = dst_v[r, pl.ds(h * 64 + k * 16, 16)]
            z = (plsc.load_gather(pd_v, [di])
                 + plsc.load_gather(ps_v, [si]) + bsc)
            l = jnp.maximum(z, 0.01 * z)
            ex = jnp.exp(l - mtop)
            den = plsc.load_gather(
                den_v,
                [lax.shift_right_logical(di, 7), lax.bitwise_and(di, 127)])
            w_v[pl.ds(k * 16, 16)] = ex / den + 1.0

        @pl.loop(0, 4)
        def _(kc):
            wch = w_v[pl.ds(kc * 16, 16)]
            for lane in range(16):
                ws = wch[lane]
                e = h * 64 + kc * 16 + lane
                for m in range(8):
                    rows_v[e, pl.ds(m * 16, 16)] = (
                        rows_v[e, pl.ds(m * 16, 16)] * ws)

    with jax.named_scope("sc_phase_c"):
        pltpu.sync_copy(src_h.at[pl.ds(base_c, 8)], src_v)
        pltpu.sync_copy(dst_h.at[pl.ds(base_c, 8)], dst_v)
        pltpu.async_copy(hv_h.at[_gidx(0, 0)], half[0], gs[0])
        pltpu.async_copy(hv_h.at[_gidx(0, 1)], half[1], gs[1])

        @pl.loop(0, ROWS_C)
        def _(q):
            r = q % 8
            for h in range(2):
                pltpu.make_async_copy(hv_h.at[_gidx(r, h)],
                                      half[h], gs[h]).wait()
                _scale_half(r, h)
                pltpu.async_copy(half[h], c_sh.at[_sidx(r, h)], ss[h],
                                 add=True)

            @pl.when(q < ROWS_C - 1)
            def _():
                @pl.when(r == 7)
                def _():
                    for h in range(2):
                        pltpu.make_async_copy(half[h], c_sh.at[_sidx(7, h)],
                                              ss[h]).wait()
                    nxt = base_c + (q // 8 + 1) * 8
                    pltpu.async_copy(src_h.at[pl.ds(nxt, 8)], src_v, gs0)
                    pltpu.async_copy(dst_h.at[pl.ds(nxt, 8)], dst_v, gs1)
                    pltpu.make_async_copy(src_h.at[pl.ds(nxt, 8)],
                                          src_v, gs0).wait()
                    pltpu.make_async_copy(dst_h.at[pl.ds(nxt, 8)],
                                          dst_v, gs1).wait()
                    for h in range(2):
                        pltpu.async_copy(hv_h.at[_gidx(0, h)], half[h], gs[h])

                @pl.when(r < 7)
                def _():
                    for h in range(2):
                        pltpu.make_async_copy(half[h], c_sh.at[_sidx(r, h)],
                                              ss[h]).wait()
                        pltpu.async_copy(hv_h.at[_gidx(r + 1, h)],
                                         half[h], gs[h])

        for h in range(2):
            pltpu.make_async_copy(half[h], c_sh.at[_sidx(7, h)], ss[h]).wait()

    with jax.named_scope("sc_copyout"):
        plsc.subcore_barrier()

        @pl.loop(0, 5)
        def _(i):
            pltpu.sync_copy(c_sh.at[pl.ds(sid * 640 + i * 128, 128)], rows_v)
            pltpu.sync_copy(
                rows_v,
                cout_h.at[pl.ds(cid * NPD + sid * 640 + i * 128, 128)])


def _sc_aggregate(pd, ps, hv_p, src_p, dst_p, b16):
    mesh = plsc.VectorSubcoreMesh(core_axis_name="core", subcore_axis_name="subcore")
    cp = pltpu.CompilerParams()
    if "needs_layout_passes" in pltpu.CompilerParams.__dataclass_fields__:
        cp = dataclasses.replace(cp, needs_layout_passes=False)
    return pl.kernel(
        _sc_body,
        compiler_params=cp,
        out_type=jax.ShapeDtypeStruct((2 * NPD, G), jnp.float32),
        mesh=mesh,
        scratch_types=[
            pltpu.VMEM((NP,), jnp.float32),
            pltpu.VMEM((NP,), jnp.float32),
            pltpu.VMEM((8, 128), jnp.int32),
            pltpu.VMEM((8, 128), jnp.int32),
            pltpu.VMEM((80, 128), jnp.float32),
            pltpu.VMEM((128, G), jnp.float32),
            pltpu.VMEM((128,), jnp.float32),
            pltpu.VMEM((1, 80), jnp.int32),
            pltpu.VMEM_SHARED((NPD, G), jnp.float32),
            pltpu.SemaphoreType.DMA,
            pltpu.SemaphoreType.DMA,
            pltpu.SemaphoreType.DMA,
            pltpu.SemaphoreType.DMA,
        ],
    )(pd, ps, hv_p, src_p, dst_p, b16)


def _gru_body(cp_ref, x_ref, wih_ref, whh_ref, bih_ref, bhh_ref, g_ref, bt_ref,
              y_ref, out_scr, sum_scr, sq_scr):
    p = pl.program_id(0)
    i = pl.program_id(1)
    blk = 1000

    @pl.when(p == 0)
    def _():
        xb = x_ref[...]
        cb = cp_ref[0] + cp_ref[1]
        ctx = jnp.where(cb > 0, cb, jnp.exp(jnp.minimum(cb, 0.0)) - 1.0)
        gi = _dot(ctx, wih_ref[...]) + bih_ref[...]
        gh = _dot(xb, whh_ref[...]) + bhh_ref[...]
        r = jax.nn.sigmoid(gi[:, :G] + gh[:, :G])
        z = jax.nn.sigmoid(gi[:, G:2 * G] + gh[:, G:2 * G])
        n = jnp.tanh(gi[:, 2 * G:] + r * gh[:, 2 * G:])
        out = jnp.maximum((1.0 - z) * n + z * xb, 0.0)
        out_scr[pl.ds(i * blk, blk), :] = out
        so = jnp.sum(out, axis=0, keepdims=True)
        sq = jnp.sum(out * out, axis=0, keepdims=True)

        @pl.when(i == 0)
        def _():
            sum_scr[...] = so
            sq_scr[...] = sq

        @pl.when(i > 0)
        def _():
            sum_scr[...] += so
            sq_scr[...] += sq

    @pl.when(p == 1)
    def _():
        mean = sum_scr[...] * (1.0 / N)
        var = sq_scr[...] * (1.0 / N) - mean * mean
        inv = jax.lax.rsqrt(var + 1e-5)
        y_ref[...] = ((out_scr[pl.ds(i * blk, blk), :] - mean) * inv
                      * g_ref[...] + bt_ref[...])


def _gru_bn(c_pair, x, wih, whh, bih, bhh, gamma, beta):
    blk = 1000
    return pl.pallas_call(
        _gru_body,
        grid=(2, N // blk),
        in_specs=[
            pl.BlockSpec((2, blk, G), lambda p, i: (0, i, 0)),
            pl.BlockSpec((blk, D), lambda p, i: (i, 0)),
            pl.BlockSpec((G, 3 * D), lambda p, i: (0, 0)),
            pl.BlockSpec((D, 3 * D), lambda p, i: (0, 0)),
            pl.BlockSpec((1, 3 * D), lambda p, i: (0, 0)),
            pl.BlockSpec((1, 3 * D), lambda p, i: (0, 0)),
            pl.BlockSpec((1, G), lambda p, i: (0, 0)),
            pl.BlockSpec((1, G), lambda p, i: (0, 0)),
        ],
        out_specs=pl.BlockSpec((blk, G), lambda p, i: (i, 0)),
        out_shape=jax.ShapeDtypeStruct((N, G), jnp.float32),
        scratch_shapes=[
            pltpu.VMEM((N, G), jnp.float32),
            pltpu.VMEM((1, G), jnp.float32),
            pltpu.VMEM((1, G), jnp.float32),
        ],
    )(c_pair, x, wih, whh, bih, bhh, gamma, beta)


def kernel(node_feats, edge_index, W_edge, b_edge, W_proj, b_proj,
           W_ih, b_ih, W_hh, b_hh, gamma, beta):
    wcat = jnp.concatenate(
        [W_proj.T,
         W_edge[0, :D][:, None],
         W_edge[0, D:][:, None],
         jnp.zeros((D, G - 2), jnp.float32)], axis=1)
    hv, pq = _prep(node_feats, wcat, b_proj.reshape(1, G))

    pd = jnp.pad(pq[:, 0], (0, NP - N))
    ps = jnp.pad(pq[:, 1], (0, NP - N))
    hv_p = jnp.pad(hv, ((0, NP - N), (0, 0)))

    pad_n = (EP - E) // 32
    pad_src = jnp.broadcast_to(
        N + (jnp.arange(pad_n, dtype=jnp.int32) % 16), (32, pad_n))
    pad_dst = jnp.broadcast_to(
        N + (jnp.arange(32, dtype=jnp.int32) % 16)[:, None], (32, pad_n))
    src_p = jnp.concatenate(
        [edge_index[0].reshape(32, E // 32), pad_src], axis=1).reshape(ER, 128)
    dst_p = jnp.concatenate(
        [edge_index[1].reshape(32, E // 32), pad_dst], axis=1).reshape(ER, 128)
    b16 = jnp.broadcast_to(b_edge, (16,)).astype(jnp.float32)

    cout = _sc_aggregate(pd, ps, hv_p, src_p, dst_p, b16)
    c_pair = cout.reshape(2, NPD, G)

    return _gru_bn(c_pair, node_feats,
                   W_ih.T, W_hh.T,
                   b_ih.reshape(1, 3 * D), b_hh.reshape(1, 3 * D),
                   gamma.reshape(1, G), beta.reshape(1, G))
